# baseline (device time: 165120 ns/iter reference)
import jax
import jax.numpy as jnp
from jax import lax
from jax.experimental import pallas as pl
from jax.experimental.pallas import tpu as pltpu

N_DEV = 4
SUBS = 8


def kernel(x, w_mat):
    m, k_per = x.shape
    _, n = w_mat.shape
    m_chunk = m // N_DEV
    n_half = n // 2
    n_hops = 2 * (N_DEV - 1)
    m_sub = m_chunk // SUBS

    def body(x_ref, w_ref, out_ref, w16_ref, comm_r_ref, comm_l_ref,
             send_sems_r, recv_sems_r, send_sems_l, recv_sems_l, store_sems):
        my = lax.axis_index("i")
        left = lax.rem(my + (N_DEV - 1), N_DEV)
        right = lax.rem(my + 1, N_DEV)

        barrier_sem = pltpu.get_barrier_semaphore()
        for nbr in (left, right):
            pl.semaphore_signal(
                barrier_sem, inc=1,
                device_id=(nbr,), device_id_type=pl.DeviceIdType.MESH,
            )
        w16_ref[...] = w_ref[...].astype(jnp.bfloat16)
        pl.semaphore_wait(barrier_sem, 2)

        comm = (comm_r_ref, comm_l_ref)
        sems = ((send_sems_r, recv_sems_r), (send_sems_l, recv_sems_l))
        peer = (right, left)

        def partial_chunk(c, half):
            xs = x_ref[pl.ds(c * m_chunk, m_chunk), :].astype(jnp.bfloat16)
            ws = w16_ref[:, pl.ds(half * n_half, n_half)]
            return jnp.dot(xs, ws, preferred_element_type=jnp.float32)

        def sub_rdma(d, h, b):
            s, r = h % 2, (h + 1) % 2
            rows = pl.ds(b * m_sub, m_sub)
            return pltpu.make_async_remote_copy(
                src_ref=comm[d].at[s, rows, :],
                dst_ref=comm[d].at[r, rows, :],
                send_sem=sems[d][0].at[h, b],
                recv_sem=sems[d][1].at[h, b],
                device_id=(peer[d],), device_id_type=pl.DeviceIdType.MESH,
            )

        def c_ring(d, i):
            return lax.rem(my + (2 * N_DEV - 1 - i), N_DEV) if d == 0 \
                else lax.rem(my + 1 + i, N_DEV)

        def start_store(d, c, slot):
            copy = pltpu.make_async_copy(
                comm[d].at[slot],
                out_ref.at[pl.ds(c * m_chunk, m_chunk),
                           pl.ds(d * n_half, n_half)],
                store_sems.at[d],
            )
            copy.start()
            return copy

        for b in range(SUBS):
            rows = pl.ds(b * m_sub, m_sub)
            for d in (0, 1):
                c = c_ring(d, 0)
                xs = x_ref[pl.ds(c * m_chunk + b * m_sub, m_sub), :].astype(
                    jnp.bfloat16)
                ws = w16_ref[:, pl.ds(d * n_half, n_half)]
                p_sub = jnp.dot(xs, ws, preferred_element_type=jnp.float32)
                comm[d][0, rows, :] = p_sub.astype(jnp.bfloat16)
                sub_rdma(d, 0, b).start()

        for h in range(1, N_DEV - 1):
            p = [partial_chunk(c_ring(d, h), d) for d in (0, 1)]
            s = h % 2
            for b in range(SUBS):
                rows = pl.ds(b * m_sub, m_sub)
                for d in (0, 1):
                    sub_rdma(d, h - 1, b).wait()
                    acc = (p[d][b * m_sub:(b + 1) * m_sub, :]
                           + comm[d][s, rows, :].astype(jnp.float32))
                    comm[d][s, rows, :] = acc.astype(jnp.bfloat16)
                    sub_rdma(d, h, b).start()

        p = [partial_chunk(my, d) for d in (0, 1)]
        last = (N_DEV - 1) % 2
        for b in range(SUBS):
            rows = pl.ds(b * m_sub, m_sub)
            for d in (0, 1):
                sub_rdma(d, N_DEV - 2, b).wait()
                red = (p[d][b * m_sub:(b + 1) * m_sub, :]
                       + comm[d][last, rows, :].astype(jnp.float32))
                y = red * (1.0 / (1.0 + jnp.exp(-red)))
                comm[d][last, rows, :] = y.astype(jnp.bfloat16)
                sub_rdma(d, N_DEV - 1, b).start()
        pending = [start_store(d, my, last) for d in (0, 1)]

        for t in range(1, N_DEV - 1):
            h = (N_DEV - 1) + t
            for b in range(SUBS):
                for d in (0, 1):
                    sub_rdma(d, h - 1, b).wait()
                    sub_rdma(d, h, b).start()
            for st in pending:
                st.wait()
            pending = [start_store(d, c_ring(d, t - 1), h % 2)
                       for d in (0, 1)]

        for st in pending:
            st.wait()
        tail = []
        for b in range(SUBS):
            rows = pl.ds(b * m_sub, m_sub)
            for d in (0, 1):
                sub_rdma(d, n_hops - 1, b).wait()
                c = c_ring(d, N_DEV - 2)
                copy = pltpu.make_async_copy(
                    comm[d].at[0, rows, :],
                    out_ref.at[pl.ds(c * m_chunk + b * m_sub, m_sub),
                               pl.ds(d * n_half, n_half)],
                    store_sems.at[d],
                )
                copy.start()
                tail.append(copy)
        for copy in tail:
            copy.wait()

    return pl.pallas_call(
        body,
        out_shape=jax.ShapeDtypeStruct((m, n), jnp.bfloat16),
        in_specs=[
            pl.BlockSpec(memory_space=pltpu.VMEM),
            pl.BlockSpec(memory_space=pltpu.VMEM),
        ],
        out_specs=pl.BlockSpec(memory_space=pl.ANY),
        scratch_shapes=[
            pltpu.VMEM((k_per, n), jnp.bfloat16),
            pltpu.VMEM((2, m_chunk, n_half), jnp.bfloat16),
            pltpu.VMEM((2, m_chunk, n_half), jnp.bfloat16),
            pltpu.SemaphoreType.DMA((n_hops, SUBS)),
            pltpu.SemaphoreType.DMA((n_hops, SUBS)),
            pltpu.SemaphoreType.DMA((n_hops, SUBS)),
            pltpu.SemaphoreType.DMA((n_hops, SUBS)),
            pltpu.SemaphoreType.DMA((2,)),
        ],
        compiler_params=pltpu.CompilerParams(
            collective_id=0,
            vmem_limit_bytes=62 * 1024 * 1024,
        ),
    )(x, w_mat)


# device time: 164839 ns/iter; 1.0017x vs baseline; 1.0017x over previous
import jax
import jax.numpy as jnp
from jax import lax
from jax.experimental import pallas as pl
from jax.experimental.pallas import tpu as pltpu

N_DEV = 4
SUBS = 4


def kernel(x, w_mat):
    m, k_per = x.shape
    _, n = w_mat.shape
    m_chunk = m // N_DEV
    n_half = n // 2
    n_hops = 2 * (N_DEV - 1)
    m_sub = m_chunk // SUBS

    def body(x_ref, w_ref, out_ref, w16_ref, comm_r_ref, comm_l_ref,
             send_sems_r, recv_sems_r, send_sems_l, recv_sems_l, store_sems):
        my = lax.axis_index("i")
        left = lax.rem(my + (N_DEV - 1), N_DEV)
        right = lax.rem(my + 1, N_DEV)

        barrier_sem = pltpu.get_barrier_semaphore()
        for nbr in (left, right):
            pl.semaphore_signal(
                barrier_sem, inc=1,
                device_id=(nbr,), device_id_type=pl.DeviceIdType.MESH,
            )
        w16_ref[...] = w_ref[...].astype(jnp.bfloat16)
        pl.semaphore_wait(barrier_sem, 2)

        comm = (comm_r_ref, comm_l_ref)
        sems = ((send_sems_r, recv_sems_r), (send_sems_l, recv_sems_l))
        peer = (right, left)

        def partial_chunk(c, half):
            xs = x_ref[pl.ds(c * m_chunk, m_chunk), :].astype(jnp.bfloat16)
            ws = w16_ref[:, pl.ds(half * n_half, n_half)]
            return jnp.dot(xs, ws, preferred_element_type=jnp.float32)

        def sub_rdma(d, h, b):
            s, r = h % 2, (h + 1) % 2
            rows = pl.ds(b * m_sub, m_sub)
            return pltpu.make_async_remote_copy(
                src_ref=comm[d].at[s, rows, :],
                dst_ref=comm[d].at[r, rows, :],
                send_sem=sems[d][0].at[h, b],
                recv_sem=sems[d][1].at[h, b],
                device_id=(peer[d],), device_id_type=pl.DeviceIdType.MESH,
            )

        def c_ring(d, i):
            return lax.rem(my + (2 * N_DEV - 1 - i), N_DEV) if d == 0 \
                else lax.rem(my + 1 + i, N_DEV)

        def start_store(d, c, slot):
            copy = pltpu.make_async_copy(
                comm[d].at[slot],
                out_ref.at[pl.ds(c * m_chunk, m_chunk),
                           pl.ds(d * n_half, n_half)],
                store_sems.at[d],
            )
            copy.start()
            return copy

        for b in range(SUBS):
            rows = pl.ds(b * m_sub, m_sub)
            for d in (0, 1):
                c = c_ring(d, 0)
                xs = x_ref[pl.ds(c * m_chunk + b * m_sub, m_sub), :].astype(
                    jnp.bfloat16)
                ws = w16_ref[:, pl.ds(d * n_half, n_half)]
                p_sub = jnp.dot(xs, ws, preferred_element_type=jnp.float32)
                comm[d][0, rows, :] = p_sub.astype(jnp.bfloat16)
                sub_rdma(d, 0, b).start()

        for h in range(1, N_DEV - 1):
            p = [partial_chunk(c_ring(d, h), d) for d in (0, 1)]
            s = h % 2
            for b in range(SUBS):
                rows = pl.ds(b * m_sub, m_sub)
                for d in (0, 1):
                    sub_rdma(d, h - 1, b).wait()
                    acc = (p[d][b * m_sub:(b + 1) * m_sub, :]
                           + comm[d][s, rows, :].astype(jnp.float32))
                    comm[d][s, rows, :] = acc.astype(jnp.bfloat16)
                    sub_rdma(d, h, b).start()

        p = [partial_chunk(my, d) for d in (0, 1)]
        last = (N_DEV - 1) % 2
        for b in range(SUBS):
            rows = pl.ds(b * m_sub, m_sub)
            for d in (0, 1):
                sub_rdma(d, N_DEV - 2, b).wait()
                red = (p[d][b * m_sub:(b + 1) * m_sub, :]
                       + comm[d][last, rows, :].astype(jnp.float32))
                y = red * (1.0 / (1.0 + jnp.exp(-red)))
                comm[d][last, rows, :] = y.astype(jnp.bfloat16)
                sub_rdma(d, N_DEV - 1, b).start()
        pending = [start_store(d, my, last) for d in (0, 1)]

        for t in range(1, N_DEV - 1):
            h = (N_DEV - 1) + t
            for b in range(SUBS):
                for d in (0, 1):
                    sub_rdma(d, h - 1, b).wait()
                    sub_rdma(d, h, b).start()
            for st in pending:
                st.wait()
            pending = [start_store(d, c_ring(d, t - 1), h % 2)
                       for d in (0, 1)]

        for st in pending:
            st.wait()
        tail = []
        for b in range(SUBS):
            rows = pl.ds(b * m_sub, m_sub)
            for d in (0, 1):
                sub_rdma(d, n_hops - 1, b).wait()
                c = c_ring(d, N_DEV - 2)
                copy = pltpu.make_async_copy(
                    comm[d].at[0, rows, :],
                    out_ref.at[pl.ds(c * m_chunk + b * m_sub, m_sub),
                               pl.ds(d * n_half, n_half)],
                    store_sems.at[d],
                )
                copy.start()
                tail.append(copy)
        for copy in tail:
            copy.wait()

    return pl.pallas_call(
        body,
        out_shape=jax.ShapeDtypeStruct((m, n), jnp.bfloat16),
        in_specs=[
            pl.BlockSpec(memory_space=pltpu.VMEM),
            pl.BlockSpec(memory_space=pltpu.VMEM),
        ],
        out_specs=pl.BlockSpec(memory_space=pl.ANY),
        scratch_shapes=[
            pltpu.VMEM((k_per, n), jnp.bfloat16),
            pltpu.VMEM((2, m_chunk, n_half), jnp.bfloat16),
            pltpu.VMEM((2, m_chunk, n_half), jnp.bfloat16),
            pltpu.SemaphoreType.DMA((n_hops, SUBS)),
            pltpu.SemaphoreType.DMA((n_hops, SUBS)),
            pltpu.SemaphoreType.DMA((n_hops, SUBS)),
            pltpu.SemaphoreType.DMA((n_hops, SUBS)),
            pltpu.SemaphoreType.DMA((2,)),
        ],
        compiler_params=pltpu.CompilerParams(
            collective_id=0,
            vmem_limit_bytes=62 * 1024 * 1024,
        ),
    )(x, w_mat)
